# Initial kernel scaffold; baseline (speedup 1.0000x reference)
import jax
import jax.numpy as jnp
from jax.experimental import pallas as pl


def kernel(preds, captions):
    raise NotImplementedError("write your pallas kernel here")


# speedup vs baseline: 1.4717x; 1.4717x over previous
import functools

import jax
import jax.numpy as jnp
from jax import lax
from jax.experimental import pallas as pl
from jax.experimental.pallas import tpu as pltpu
from jax.experimental.pallas import tpu_sc as plsc

_VOCAB = 100000
_NC = 2
_NS = 16
_NW = _NC * _NS
_VP = 100352
_SLICE = _VP // _NS

_N_PRED = 16384 * 50
_N_CAPT = 16384 * 200
_CHUNK = 6400
_PRED_PER_W = _N_PRED // _NW
_CAPT_PER_W = _N_CAPT // _NW


def _sc_hist_body(preds_hbm, capt_hbm, pred_out, capt_out,
                  pred_acc, capt_acc, idx_buf, ones_buf, zbuf):
  c = lax.axis_index("c")
  s = lax.axis_index("s")
  w = c * _NS + s

  def fill_z(i, carry):
    zbuf[pl.ds(i * 16, 16)] = jnp.zeros((16,), jnp.int32)
    return carry

  def fill_o(i, carry):
    ones_buf[pl.ds(i * 16, 16)] = jnp.ones((16,), jnp.int32)
    return carry

  lax.fori_loop(0, _SLICE // 16, fill_z, 0)
  lax.fori_loop(0, _CHUNK // 16, fill_o, 0)

  pltpu.sync_copy(zbuf, pred_acc.at[pl.ds(s * _SLICE, _SLICE)])
  pltpu.sync_copy(zbuf, capt_acc.at[pl.ds(s * _SLICE, _SLICE)])
  plsc.subcore_barrier()

  def pred_step(j, carry):
    base = w * _PRED_PER_W + j * _CHUNK
    pltpu.sync_copy(preds_hbm.at[pl.ds(base, _CHUNK)], idx_buf)
    pltpu.sync_copy(ones_buf, pred_acc.at[idx_buf], add=True)
    return carry

  def capt_step(j, carry):
    base = w * _CAPT_PER_W + j * _CHUNK
    pltpu.sync_copy(capt_hbm.at[pl.ds(base, _CHUNK)], idx_buf)
    pltpu.sync_copy(ones_buf, capt_acc.at[idx_buf], add=True)
    return carry

  lax.fori_loop(0, _PRED_PER_W // _CHUNK, pred_step, 0)
  lax.fori_loop(0, _CAPT_PER_W // _CHUNK, capt_step, 0)
  plsc.subcore_barrier()

  off = c * _VP + s * _SLICE
  pltpu.sync_copy(pred_acc.at[pl.ds(s * _SLICE, _SLICE)],
                  pred_out.at[pl.ds(off, _SLICE)])
  pltpu.sync_copy(capt_acc.at[pl.ds(s * _SLICE, _SLICE)],
                  capt_out.at[pl.ds(off, _SLICE)])


_sc_hist = pl.kernel(
    _sc_hist_body,
    out_type=(
        jax.ShapeDtypeStruct((_NC * _VP,), jnp.int32),
        jax.ShapeDtypeStruct((_NC * _VP,), jnp.int32),
    ),
    mesh=plsc.VectorSubcoreMesh(core_axis_name="c", subcore_axis_name="s"),
    scratch_types=(
        pltpu.VMEM_SHARED((_VP,), jnp.int32),
        pltpu.VMEM_SHARED((_VP,), jnp.int32),
        pltpu.VMEM((_CHUNK,), jnp.int32),
        pltpu.VMEM((_CHUNK,), jnp.int32),
        pltpu.VMEM((_SLICE,), jnp.int32),
    ),
)


def _tc_merge_body(ph_ref, ch_ref, out_ref):
  n_pred = jnp.sum((ph_ref[0] + ph_ref[1]) > 0).astype(jnp.float32)
  n_capt = jnp.sum((ch_ref[0] + ch_ref[1]) > 0).astype(jnp.float32)
  out_ref[0, 0] = jnp.where(
      n_capt > 0, n_pred / jnp.maximum(n_capt, 1.0), jnp.float32(0.0))


@jax.jit
def kernel(preds, captions):
  pf = preds.reshape(-1)
  cf = captions.reshape(-1)
  ph_flat, ch_flat = _sc_hist(pf, cf)
  ph = ph_flat.reshape(_NC, _VP)
  ch = ch_flat.reshape(_NC, _VP)
  ratio = pl.pallas_call(
      _tc_merge_body,
      out_shape=jax.ShapeDtypeStruct((1, 1), jnp.float32),
      in_specs=[
          pl.BlockSpec(memory_space=pltpu.VMEM),
          pl.BlockSpec(memory_space=pltpu.VMEM),
      ],
      out_specs=pl.BlockSpec(memory_space=pltpu.SMEM),
  )(ph, ch)
  return ratio[0, 0]


# speedup vs baseline: 1.6543x; 1.1241x over previous
import jax
import jax.numpy as jnp
from jax import lax
from jax.experimental import pallas as pl
from jax.experimental.pallas import tpu as pltpu
from jax.experimental.pallas import tpu_sc as plsc

_VOCAB = 100000
_NC = 2
_NS = 16
_NW = _NC * _NS
_VP = 100352
_SLICE = _VP // _NS

_N_PRED = 16384 * 50
_N_CAPT = 16384 * 200
_CHUNK = 6400
_PRED_PER_W = _N_PRED // _NW
_CAPT_PER_W = _N_CAPT // _NW


def _sc_hist_body(preds_hbm, capt_hbm, pred_out, capt_out,
                  pred_acc, capt_acc, idx0, idx1, ones_buf, zbuf,
                  sem0, sem1):
  c = lax.axis_index("c")
  s = lax.axis_index("s")
  w = c * _NS + s

  def fill_z(i, carry):
    zbuf[pl.ds(i * 16, 16)] = jnp.zeros((16,), jnp.int32)
    return carry

  def fill_o(i, carry):
    ones_buf[pl.ds(i * 16, 16)] = jnp.ones((16,), jnp.int32)
    return carry

  lax.fori_loop(0, _SLICE // 16, fill_z, 0)
  lax.fori_loop(0, _CHUNK // 16, fill_o, 0)

  pltpu.sync_copy(zbuf, pred_acc.at[pl.ds(s * _SLICE, _SLICE)])
  pltpu.sync_copy(zbuf, capt_acc.at[pl.ds(s * _SLICE, _SLICE)])
  plsc.subcore_barrier()

  bufs = (idx0, idx1)
  sems = (sem0, sem1)

  def scatter_input(hbm, acc, n_chunks, per_w):
    cps = [None] * n_chunks
    cps[0] = pltpu.async_copy(
        hbm.at[pl.ds(w * per_w, _CHUNK)], bufs[0], sems[0])
    for j in range(n_chunks):
      if j + 1 < n_chunks:
        base = w * per_w + (j + 1) * _CHUNK
        cps[j + 1] = pltpu.async_copy(
            hbm.at[pl.ds(base, _CHUNK)], bufs[(j + 1) % 2], sems[(j + 1) % 2])
      cps[j].wait()
      pltpu.sync_copy(ones_buf, acc.at[bufs[j % 2]], add=True)

  scatter_input(preds_hbm, pred_acc, _PRED_PER_W // _CHUNK, _PRED_PER_W)
  scatter_input(capt_hbm, capt_acc, _CAPT_PER_W // _CHUNK, _CAPT_PER_W)
  plsc.subcore_barrier()

  off = c * _VP + s * _SLICE
  pltpu.sync_copy(pred_acc.at[pl.ds(s * _SLICE, _SLICE)],
                  pred_out.at[pl.ds(off, _SLICE)])
  pltpu.sync_copy(capt_acc.at[pl.ds(s * _SLICE, _SLICE)],
                  capt_out.at[pl.ds(off, _SLICE)])


_sc_hist = pl.kernel(
    _sc_hist_body,
    out_type=(
        jax.ShapeDtypeStruct((_NC * _VP,), jnp.int32),
        jax.ShapeDtypeStruct((_NC * _VP,), jnp.int32),
    ),
    mesh=plsc.VectorSubcoreMesh(core_axis_name="c", subcore_axis_name="s"),
    scratch_types=(
        pltpu.VMEM_SHARED((_VP,), jnp.int32),
        pltpu.VMEM_SHARED((_VP,), jnp.int32),
        pltpu.VMEM((_CHUNK,), jnp.int32),
        pltpu.VMEM((_CHUNK,), jnp.int32),
        pltpu.VMEM((_CHUNK,), jnp.int32),
        pltpu.VMEM((_SLICE,), jnp.int32),
        pltpu.SemaphoreType.DMA,
        pltpu.SemaphoreType.DMA,
    ),
)


def _tc_merge_body(ph_ref, ch_ref, out_ref):
  n_pred = jnp.sum((ph_ref[0] + ph_ref[1]) > 0).astype(jnp.float32)
  n_capt = jnp.sum((ch_ref[0] + ch_ref[1]) > 0).astype(jnp.float32)
  out_ref[0, 0] = jnp.where(
      n_capt > 0, n_pred / jnp.maximum(n_capt, 1.0), jnp.float32(0.0))


@jax.jit
def kernel(preds, captions):
  pf = preds.reshape(-1)
  cf = captions.reshape(-1)
  ph_flat, ch_flat = _sc_hist(pf, cf)
  ph = ph_flat.reshape(_NC, _VP)
  ch = ch_flat.reshape(_NC, _VP)
  ratio = pl.pallas_call(
      _tc_merge_body,
      out_shape=jax.ShapeDtypeStruct((1, 1), jnp.float32),
      in_specs=[
          pl.BlockSpec(memory_space=pltpu.VMEM),
          pl.BlockSpec(memory_space=pltpu.VMEM),
      ],
      out_specs=pl.BlockSpec(memory_space=pltpu.SMEM),
  )(ph, ch)
  return ratio[0, 0]
